# baseline (speedup 1.0000x reference)
import math

import jax
import jax.numpy as jnp
from jax import lax
from jax.experimental import pallas as pl
from jax.experimental.pallas import tpu as pltpu

_PI = 3.14159265
_T1 = math.cos(1.0 * _PI / 4.0)
_T2 = math.cos(2.0 * _PI / 4.0)
_T3 = math.cos(3.0 * _PI / 4.0)


def _angle_linear_body(x_ref, w_ref, cos_ref, phi_ref, wn_ref):
    i = pl.program_id(1)

    @pl.when(i == 0)
    def _prepare_weights():
        wf = w_ref[...]
        sw = jnp.sum(wf * wf, axis=0, keepdims=True)
        inv_w = lax.rsqrt(jnp.maximum(sw, 1e-30))
        wn_ref[...] = (wf * inv_w).astype(jnp.bfloat16)

    xf = x_ref[...]
    sx = jnp.sum(xf * xf, axis=1, keepdims=True)
    inv_x = lax.rsqrt(jnp.maximum(sx, 1e-30))
    xlen = sx * inv_x
    xn = (xf * inv_x).astype(jnp.bfloat16)

    dot = jnp.dot(xn, wn_ref[...], preferred_element_type=jnp.float32)
    c = jnp.clip(dot, -1.0, 1.0)

    c2 = c * c
    p = (8.0 * c2 - 8.0) * c2

    m1 = c <= _T1
    m2 = c <= _T2
    m3 = c <= _T3
    parity = jnp.logical_xor(jnp.logical_xor(m1, m2), m3)
    sp = jnp.where(parity, -p, p)
    qa = jnp.where(m1, jnp.float32(-3.0), jnp.float32(1.0))
    q = jnp.where(m3, qa - 4.0, qa)
    phi = sp + q

    cos_ref[...] = c * xlen
    phi_ref[...] = phi * xlen


def kernel(x, weight):
    B, D = x.shape
    D2, N = weight.shape
    assert D == D2

    tn = 2048 if N % 2048 == 0 else min(N, 2048)
    bb = 128 if B % 128 == 0 else B
    grid = (pl.cdiv(N, tn), pl.cdiv(B, bb))

    cos_t, phi_t = pl.pallas_call(
        _angle_linear_body,
        out_shape=(
            jax.ShapeDtypeStruct((B, N), x.dtype),
            jax.ShapeDtypeStruct((B, N), x.dtype),
        ),
        grid=grid,
        in_specs=[
            pl.BlockSpec((bb, D), lambda j, i: (i, 0)),
            pl.BlockSpec((D, tn), lambda j, i: (0, j)),
        ],
        out_specs=(
            pl.BlockSpec((bb, tn), lambda j, i: (i, j)),
            pl.BlockSpec((bb, tn), lambda j, i: (i, j)),
        ),
        scratch_shapes=[pltpu.VMEM((D, tn), jnp.bfloat16)],
        compiler_params=pltpu.CompilerParams(
            dimension_semantics=("parallel", "arbitrary"),
            vmem_limit_bytes=40 << 20,
        ),
    )(x, weight)
    return cos_t, phi_t


# speedup vs baseline: 1.0905x; 1.0905x over previous
import math

import jax
import jax.numpy as jnp
from jax import lax
from jax.experimental import pallas as pl
from jax.experimental.pallas import tpu as pltpu

_PI = 3.14159265
_T1 = math.cos(1.0 * _PI / 4.0)
_T2 = math.cos(2.0 * _PI / 4.0)
_T3 = math.cos(3.0 * _PI / 4.0)


def _angle_linear_body(x_ref, w_ref, cos_ref, phi_ref, wn_ref):
    i = pl.program_id(1)

    @pl.when(i == 0)
    def _prepare_weights():
        wf = w_ref[...]
        sw = jnp.sum(wf * wf, axis=0, keepdims=True)
        inv_w = lax.rsqrt(jnp.maximum(sw, 1e-30))
        wn_ref[...] = (wf * inv_w).astype(jnp.bfloat16)

    xf = x_ref[...]
    sx = jnp.sum(xf * xf, axis=1, keepdims=True)
    inv_x = lax.rsqrt(jnp.maximum(sx, 1e-30))
    xlen = sx * inv_x
    xn = (xf * inv_x).astype(jnp.bfloat16)

    dot = jnp.dot(xn, wn_ref[...], preferred_element_type=jnp.float32)
    c = jnp.clip(dot, -1.0, 1.0)

    c2 = c * c
    p = (8.0 * c2 - 8.0) * c2

    m1 = c <= _T1
    m2 = c <= _T2
    m3 = c <= _T3
    parity = jnp.logical_xor(jnp.logical_xor(m1, m2), m3)
    sp = jnp.where(parity, -p, p)
    qa = jnp.where(m1, jnp.float32(-3.0), jnp.float32(1.0))
    q = jnp.where(m3, qa - 4.0, qa)
    phi = sp + q

    cos_ref[...] = c * xlen
    phi_ref[...] = phi * xlen


def kernel(x, weight):
    B, D = x.shape
    D2, N = weight.shape
    assert D == D2

    tn = 2048 if N % 2048 == 0 else min(N, 2048)
    bb = 256 if B % 256 == 0 else B
    grid = (pl.cdiv(N, tn), pl.cdiv(B, bb))

    cos_t, phi_t = pl.pallas_call(
        _angle_linear_body,
        out_shape=(
            jax.ShapeDtypeStruct((B, N), x.dtype),
            jax.ShapeDtypeStruct((B, N), x.dtype),
        ),
        grid=grid,
        in_specs=[
            pl.BlockSpec((bb, D), lambda j, i: (i, 0)),
            pl.BlockSpec((D, tn), lambda j, i: (0, j)),
        ],
        out_specs=(
            pl.BlockSpec((bb, tn), lambda j, i: (i, j)),
            pl.BlockSpec((bb, tn), lambda j, i: (i, j)),
        ),
        scratch_shapes=[pltpu.VMEM((D, tn), jnp.bfloat16)],
        compiler_params=pltpu.CompilerParams(
            dimension_semantics=("parallel", "arbitrary"),
            vmem_limit_bytes=40 << 20,
        ),
    )(x, weight)
    return cos_t, phi_t


# speedup vs baseline: 1.4430x; 1.3233x over previous
import math

import jax
import jax.numpy as jnp
from jax import lax
from jax.experimental import pallas as pl
from jax.experimental.pallas import tpu as pltpu

_PI = 3.14159265
_T1 = math.cos(1.0 * _PI / 4.0)
_T2 = math.cos(2.0 * _PI / 4.0)
_T3 = math.cos(3.0 * _PI / 4.0)


def _angle_linear_body(x_ref, w_ref, cos_ref, phi_ref):
    xf = x_ref[...]
    sx = jnp.sum(xf * xf, axis=1, keepdims=True)
    inv_x = lax.rsqrt(jnp.maximum(sx, 1e-30))
    xlen = sx * inv_x
    xn = (xf * inv_x).astype(jnp.bfloat16)

    wf = w_ref[...]
    sw = jnp.sum(wf * wf, axis=0, keepdims=True)
    inv_w = lax.rsqrt(jnp.maximum(sw, 1e-30))
    wn = (wf * inv_w).astype(jnp.bfloat16)

    dot = jnp.dot(xn, wn, preferred_element_type=jnp.float32)
    c = jnp.clip(dot, -1.0, 1.0)

    c2 = c * c
    p = (8.0 * c2 - 8.0) * c2

    m1 = c <= _T1
    m2 = c <= _T2
    m3 = c <= _T3
    parity = jnp.logical_xor(jnp.logical_xor(m1, m2), m3)
    sp = jnp.where(parity, -p, p)
    qa = jnp.where(m1, jnp.float32(-3.0), jnp.float32(1.0))
    q = jnp.where(m3, qa - 4.0, qa)
    phi = sp + q

    cos_ref[...] = c * xlen
    phi_ref[...] = phi * xlen


def kernel(x, weight):
    B, D = x.shape
    D2, N = weight.shape
    assert D == D2

    tn = 2048 if N % 2048 == 0 else min(N, 2048)
    grid = (pl.cdiv(N, tn),)

    cos_t, phi_t = pl.pallas_call(
        _angle_linear_body,
        out_shape=(
            jax.ShapeDtypeStruct((B, N), x.dtype),
            jax.ShapeDtypeStruct((B, N), x.dtype),
        ),
        grid=grid,
        in_specs=[
            pl.BlockSpec((B, D), lambda j: (0, 0)),
            pl.BlockSpec((D, tn), lambda j: (0, j)),
        ],
        out_specs=(
            pl.BlockSpec((B, tn), lambda j: (0, j)),
            pl.BlockSpec((B, tn), lambda j: (0, j)),
        ),
        compiler_params=pltpu.CompilerParams(
            dimension_semantics=("parallel",),
            vmem_limit_bytes=48 << 20,
        ),
    )(x, weight)
    return cos_t, phi_t
